# baseline (speedup 1.0000x reference)
import jax
import jax.numpy as jnp
from jax.experimental import pallas as pl

SEQ = 21


def _body(x_ref, t_ref, o_ref):
    row = t_ref[0, 0, :]
    norm = jnp.sqrt(jnp.sum(row * row))
    scale = jnp.where(norm > 1.0, 1.0 / (norm + 1e-7), 1.0)
    o_ref[...] = x_ref[...] + (row * scale)[None, None, :]


def kernel(x, table, num_people):
    del num_people
    N, T, D = x.shape
    V = table.shape[0]
    grid = (N,)
    out = pl.pallas_call(
        _body,
        grid=grid,
        in_specs=[
            pl.BlockSpec((1, T, D), lambda i: (i, 0, 0)),
            pl.BlockSpec((1, 1, D), lambda i: (i // SEQ, 0, 0)),
        ],
        out_specs=pl.BlockSpec((1, T, D), lambda i: (i, 0, 0)),
        out_shape=jax.ShapeDtypeStruct((N, T, D), x.dtype),
    )(x, table.reshape(V, 1, D))
    return out


# speedup vs baseline: 3.9909x; 3.9909x over previous
import jax
import jax.numpy as jnp
from jax.experimental import pallas as pl

SEQ = 21


def _body(x_ref, t_ref, o_ref):
    row = t_ref[0, 0, :]
    norm = jnp.sqrt(jnp.sum(row * row))
    scale = jnp.where(norm > 1.0, 1.0 / (norm + 1e-7), 1.0)
    o_ref[...] = x_ref[...] + (row * scale)[None, None, :]


def kernel(x, table, num_people):
    del num_people
    N, T, D = x.shape
    V = table.shape[0]
    n = N // SEQ
    grid = (n,)
    out = pl.pallas_call(
        _body,
        grid=grid,
        in_specs=[
            pl.BlockSpec((SEQ, T, D), lambda i: (i, 0, 0)),
            pl.BlockSpec((1, 1, D), lambda i: (i, 0, 0)),
        ],
        out_specs=pl.BlockSpec((SEQ, T, D), lambda i: (i, 0, 0)),
        out_shape=jax.ShapeDtypeStruct((N, T, D), x.dtype),
    )(x, table.reshape(V, 1, D))
    return out
